# baseline (speedup 1.0000x reference)
import functools
import jax
import jax.numpy as jnp
from jax import lax
from jax.experimental import pallas as pl
from jax.experimental.pallas import tpu as pltpu
from jax.experimental.pallas import tpu_sc as plsc

_sc_params = pltpu.CompilerParams(
    needs_layout_passes=False,
    use_tc_tiling_on_sc=False,
)

N = 10000
E = 320000
D = 128
OUT = 128
HALF = OUT // 2
WA = 80
NEG_SLOPE = 0.2

NC = 2
NS = 16
NW = NC * NS
EPT = E // NW

CA = 2000
NCA = EPT // CA

SUB = 80
NST = EPT // SUB
GPS = SUB // 16
NBUF = 4

NPAD = 10112
ROWS_PER_TILE = NPAD // NS

_mesh = plsc.VectorSubcoreMesh(core_axis_name="c", subcore_axis_name="s")


def _feat_body(x_ref, wt_ref, alr_ref, elr_ref, fa_ref, fb_ref):
    wlr = jnp.dot(wt_ref[...], alr_ref[...], preferred_element_type=jnp.float32)
    elr_ref[...] = jnp.dot(x_ref[...], wlr, preferred_element_type=jnp.float32)
    f = jnp.dot(x_ref[...], wt_ref[...], preferred_element_type=jnp.float32)
    blk = f.shape[0]
    ones = jnp.ones((blk, 1), jnp.float32)
    zpad = jnp.zeros((blk, WA - HALF - 1), jnp.float32)
    fa_ref[...] = jnp.concatenate([f[:, :HALF], ones, zpad], axis=1)
    fb_ref[...] = jnp.concatenate([f[:, HALF:], ones, zpad], axis=1)


def _final_body(pfa_ref, pfb_ref, out_ref):
    fa = pfa_ref[0, :, :HALF] + pfa_ref[1, :, :HALF]
    fb = pfb_ref[0, :, :HALF] + pfb_ref[1, :, :HALF]
    es = pfa_ref[0, :, HALF:HALF + 1] + pfa_ref[1, :, HALF:HALF + 1]
    out_ref[...] = jnp.concatenate([fa, fb], axis=1) / (es + 1e-9)


@functools.partial(
    pl.kernel,
    out_type=jax.ShapeDtypeStruct((E,), jnp.float32),
    mesh=_mesh,
    scratch_types=[
        pltpu.VMEM((N,), jnp.float32),
        pltpu.VMEM((N,), jnp.float32),
        pltpu.VMEM((CA,), jnp.int32),
        pltpu.VMEM((CA,), jnp.int32),
        pltpu.VMEM((CA,), jnp.float32),
    ],
    compiler_params=_sc_params,
)
def _edge_logits(el_hbm, er_hbm, src_hbm, dst_hbm, ex_hbm,
                 elv, erv, srcb, dstb, eb):
    wid = lax.axis_index("s") * NC + lax.axis_index("c")
    pltpu.sync_copy(el_hbm, elv)
    pltpu.sync_copy(er_hbm, erv)

    base_w = wid * EPT

    for ci in range(NCA):
        base = base_w + ci * CA
        pltpu.sync_copy(src_hbm.at[pl.ds(base, CA)], srcb)
        pltpu.sync_copy(dst_hbm.at[pl.ds(base, CA)], dstb)

        @pl.loop(0, CA // 16)
        def _(g):
            s16 = srcb[pl.ds(g * 16, 16)]
            d16 = dstb[pl.ds(g * 16, 16)]
            ev = plsc.load_gather(elv, [s16]) + plsc.load_gather(erv, [d16])
            ev = jnp.where(ev > 0, ev, NEG_SLOPE * ev)
            eb[pl.ds(g * 16, 16)] = jnp.exp(ev)

        pltpu.sync_copy(eb, ex_hbm.at[pl.ds(base, CA)])


def _make_aggregate_kernel(width):
    out_type = jax.ShapeDtypeStruct((NC, NPAD, width), jnp.float32)
    scratch = [
        pltpu.VMEM((NST, SUB), jnp.int32),
        pltpu.VMEM((EPT,), jnp.int32),
        pltpu.VMEM((EPT,), jnp.float32),
        pltpu.VMEM((NBUF, SUB, width), jnp.float32),
        pltpu.VMEM_SHARED((NPAD, width), jnp.float32),
        pltpu.SemaphoreType.DMA,
        pltpu.SemaphoreType.DMA,
        pltpu.SemaphoreType.DMA,
        pltpu.SemaphoreType.DMA,
        pltpu.SemaphoreType.DMA,
    ]

    def body(feat_hbm, ex_hbm, src_hbm, dst3_hbm, zf_hbm, pf_hbm,
             dstb, srcv, exv, rows, accf, semg, sa0, sa1, sa2, sa3):
        semas = (sa0, sa1, sa2, sa3)

        cid = lax.axis_index("c")
        sid = lax.axis_index("s")
        wid = sid * NC + cid
        base_w = wid * EPT

        pltpu.sync_copy(dst3_hbm.at[wid], dstb)
        pltpu.sync_copy(src_hbm.at[pl.ds(base_w, EPT)], srcv)
        pltpu.sync_copy(ex_hbm.at[pl.ds(base_w, EPT)], exv)

        row0 = sid * ROWS_PER_TILE
        pltpu.sync_copy(zf_hbm, accf.at[pl.ds(row0, ROWS_PER_TILE)])
        plsc.subcore_barrier()

        def fire_gather(j, b):
            pltpu.async_copy(
                feat_hbm.at[srcv.at[pl.ds(j * SUB, SUB)]],
                rows.at[b], semg)

        def wait_gather():
            pltpu.make_async_copy(
                feat_hbm.at[srcv.at[pl.ds(0, SUB)]],
                rows.at[0], semg).wait()

        def fire_adds(j, b):
            pltpu.async_copy(
                rows.at[b], accf.at[dstb.at[j]], semas[b], add=True)

        def wait_adds(b):
            pltpu.make_async_copy(
                rows.at[b], accf.at[dstb.at[0]], semas[b]).wait()

        def compute(j, b):
            @plsc.parallel_loop(0, GPS, step=1)
            def _(g):
                ex = exv[pl.ds(j * SUB + g * 16, 16)]
                for r in range(16):
                    av = jnp.broadcast_to(ex[r], (16,))
                    row = g * 16 + r
                    for k in range(width // 16):
                        sl = pl.ds(k * 16, 16)
                        rows[b, row, sl] = rows[b, row, sl] * av

        def slot(j, b, bn, full):
            wait_gather()
            compute(j, b)
            fire_adds(j, b)
            if full:
                wait_adds(bn)
            fire_gather(j + 2, bn)

        fire_gather(0, 0)
        fire_gather(1, 1)
        slot(0, 0, 2, False)
        slot(1, 1, 3, False)
        slot(2, 2, 0, True)
        slot(3, 3, 1, True)

        @pl.loop(NBUF, NST - 5, step=NBUF)
        def _(j0):
            slot(j0 + 0, 0, 2, True)
            slot(j0 + 1, 1, 3, True)
            slot(j0 + 2, 2, 0, True)
            slot(j0 + 3, 3, 1, True)

        slot(NST - 5, 0, 2, True)
        slot(NST - 4, 1, 3, True)
        slot(NST - 3, 2, 0, True)

        wait_gather()
        compute(NST - 2, 3)
        fire_adds(NST - 2, 3)
        wait_gather()
        compute(NST - 1, 0)
        fire_adds(NST - 1, 0)

        wait_adds(1)
        wait_adds(2)
        wait_adds(3)
        wait_adds(0)

        plsc.subcore_barrier()
        pltpu.sync_copy(accf.at[pl.ds(row0, ROWS_PER_TILE)],
                        pf_hbm.at[cid, pl.ds(row0, ROWS_PER_TILE)])

    return pl.kernel(
        body,
        out_type=out_type,
        mesh=_mesh,
        scratch_types=scratch,
        compiler_params=_sc_params,
    )


_aggregate = _make_aggregate_kernel(WA)


@jax.jit
def kernel(x, edge_index, W, attn_l, attn_r):
    src = edge_index[0]
    dst = edge_index[1]
    wt = W.T
    alr = jnp.stack([attn_l.reshape(OUT), attn_r.reshape(OUT)], axis=1)

    elr, feat_a, feat_b = pl.pallas_call(
        _feat_body,
        grid=(10,),
        in_specs=[
            pl.BlockSpec((1000, D), lambda i: (i, 0)),
            pl.BlockSpec((D, OUT), lambda i: (0, 0)),
            pl.BlockSpec((OUT, 2), lambda i: (0, 0)),
        ],
        out_specs=[
            pl.BlockSpec((1000, 2), lambda i: (i, 0)),
            pl.BlockSpec((1000, WA), lambda i: (i, 0)),
            pl.BlockSpec((1000, WA), lambda i: (i, 0)),
        ],
        out_shape=[
            jax.ShapeDtypeStruct((N, 2), jnp.float32),
            jax.ShapeDtypeStruct((N, WA), jnp.float32),
            jax.ShapeDtypeStruct((N, WA), jnp.float32),
        ],
    )(x, wt, alr)

    el = elr[:, 0]
    er = elr[:, 1]
    eexp = _edge_logits(el, er, src, dst)

    dst3 = dst.reshape(NW, NST, SUB)
    zf = jnp.zeros((ROWS_PER_TILE, WA), jnp.float32)
    pfa = _aggregate(feat_a, eexp, src, dst3, zf)
    pfb = _aggregate(feat_b, eexp, src, dst3, zf)

    rst = pl.pallas_call(
        _final_body,
        grid=(10,),
        in_specs=[
            pl.BlockSpec((NC, 1000, WA), lambda i: (0, i, 0)),
            pl.BlockSpec((NC, 1000, WA), lambda i: (0, i, 0)),
        ],
        out_specs=pl.BlockSpec((1000, OUT), lambda i: (i, 0)),
        out_shape=jax.ShapeDtypeStruct((N, OUT), jnp.float32),
    )(pfa, pfb)
    return rst.reshape(N, 1, OUT)


# speedup vs baseline: 1.0431x; 1.0431x over previous
import functools
import jax
import jax.numpy as jnp
from jax import lax
from jax.experimental import pallas as pl
from jax.experimental.pallas import tpu as pltpu
from jax.experimental.pallas import tpu_sc as plsc

_sc_params = pltpu.CompilerParams(
    needs_layout_passes=False,
    use_tc_tiling_on_sc=False,
)

N = 10000
E = 320000
D = 128
OUT = 128
HALF = OUT // 2
WA = 80
NEG_SLOPE = 0.2

NC = 2
NS = 16
NW = NC * NS
EPT = E // NW

CA = 2000
NCA = EPT // CA

SUB = 80
NST = EPT // SUB
GPS = SUB // 16
NBUF = 4

NPAD = 10112
ROWS_PER_TILE = NPAD // NS

_mesh = plsc.VectorSubcoreMesh(core_axis_name="c", subcore_axis_name="s")


def _feat_body(x_ref, wt_ref, alr_ref, elr_ref, fa_ref, fb_ref):
    wlr = jnp.dot(wt_ref[...], alr_ref[...], preferred_element_type=jnp.float32)
    elr_ref[...] = jnp.dot(x_ref[...], wlr, preferred_element_type=jnp.float32)
    f = jnp.dot(x_ref[...], wt_ref[...], preferred_element_type=jnp.float32)
    blk = f.shape[0]
    ones = jnp.ones((blk, 1), jnp.float32)
    zpad = jnp.zeros((blk, WA - HALF - 1), jnp.float32)
    fa_ref[...] = jnp.concatenate([f[:, :HALF], ones, zpad], axis=1)
    fb_ref[...] = jnp.concatenate([f[:, HALF:], ones, zpad], axis=1)


def _final_body(pfa_ref, pfb_ref, out_ref):
    fa = pfa_ref[0, :, :HALF] + pfa_ref[1, :, :HALF]
    fb = pfb_ref[0, :, :HALF] + pfb_ref[1, :, :HALF]
    es = pfa_ref[0, :, HALF:HALF + 1] + pfa_ref[1, :, HALF:HALF + 1]
    out_ref[...] = jnp.concatenate([fa, fb], axis=1) / (es + 1e-9)


@functools.partial(
    pl.kernel,
    out_type=jax.ShapeDtypeStruct((E,), jnp.float32),
    mesh=_mesh,
    scratch_types=[
        pltpu.VMEM((N,), jnp.float32),
        pltpu.VMEM((N,), jnp.float32),
        pltpu.VMEM((CA,), jnp.int32),
        pltpu.VMEM((CA,), jnp.int32),
        pltpu.VMEM((CA,), jnp.float32),
    ],
    compiler_params=_sc_params,
)
def _edge_logits(el_hbm, er_hbm, src_hbm, dst_hbm, ex_hbm,
                 elv, erv, srcb, dstb, eb):
    wid = lax.axis_index("s") * NC + lax.axis_index("c")
    pltpu.sync_copy(el_hbm, elv)
    pltpu.sync_copy(er_hbm, erv)

    base_w = wid * EPT

    for ci in range(NCA):
        base = base_w + ci * CA
        pltpu.sync_copy(src_hbm.at[pl.ds(base, CA)], srcb)
        pltpu.sync_copy(dst_hbm.at[pl.ds(base, CA)], dstb)

        @pl.loop(0, CA // 16)
        def _(g):
            s16 = srcb[pl.ds(g * 16, 16)]
            d16 = dstb[pl.ds(g * 16, 16)]
            ev = plsc.load_gather(elv, [s16]) + plsc.load_gather(erv, [d16])
            ev = jnp.where(ev > 0, ev, NEG_SLOPE * ev)
            eb[pl.ds(g * 16, 16)] = jnp.exp(ev)

        pltpu.sync_copy(eb, ex_hbm.at[pl.ds(base, CA)])


def _make_aggregate_kernel(width):
    out_type = jax.ShapeDtypeStruct((NC, NPAD, width), jnp.float32)
    scratch = [
        pltpu.VMEM((NST, SUB), jnp.int32),
        pltpu.VMEM((EPT,), jnp.int32),
        pltpu.VMEM((EPT,), jnp.float32),
        pltpu.VMEM((NBUF, SUB, width), jnp.float32),
        pltpu.VMEM_SHARED((NPAD, width), jnp.float32),
        pltpu.SemaphoreType.DMA,
        pltpu.SemaphoreType.DMA,
        pltpu.SemaphoreType.DMA,
        pltpu.SemaphoreType.DMA,
        pltpu.SemaphoreType.DMA,
    ]

    def body(feat_hbm, ex_hbm, src_hbm, dst3_hbm, zf_hbm, pf_hbm,
             dstb, srcv, exv, rows, accf, semg, sa0, sa1, sa2, sa3):
        semas = (sa0, sa1, sa2, sa3)

        cid = lax.axis_index("c")
        sid = lax.axis_index("s")
        wid = sid * NC + cid
        base_w = wid * EPT

        pltpu.sync_copy(dst3_hbm.at[wid], dstb)
        pltpu.sync_copy(src_hbm.at[pl.ds(base_w, EPT)], srcv)
        pltpu.sync_copy(ex_hbm.at[pl.ds(base_w, EPT)], exv)

        row0 = sid * ROWS_PER_TILE
        pltpu.sync_copy(zf_hbm, accf.at[pl.ds(row0, ROWS_PER_TILE)])
        plsc.subcore_barrier()

        def fire_gather(j, b):
            pltpu.async_copy(
                feat_hbm.at[srcv.at[pl.ds(j * SUB, SUB)]],
                rows.at[b], semg)

        def wait_gather():
            pltpu.make_async_copy(
                feat_hbm.at[srcv.at[pl.ds(0, SUB)]],
                rows.at[0], semg).wait()

        def fire_adds(j, b):
            pltpu.async_copy(
                rows.at[b], accf.at[dstb.at[j]], semas[b], add=True)

        def wait_adds(b):
            pltpu.make_async_copy(
                rows.at[b], accf.at[dstb.at[0]], semas[b]).wait()

        def compute(j, b):
            @pl.loop(0, GPS)
            def _(g):
                ex = exv[pl.ds(j * SUB + g * 16, 16)]
                for r in range(16):
                    av = jnp.broadcast_to(ex[r], (16,))
                    row = g * 16 + r
                    for k in range(width // 16):
                        sl = pl.ds(k * 16, 16)
                        rows[b, row, sl] = rows[b, row, sl] * av

        def slot(j, b, bn, full):
            wait_gather()
            compute(j, b)
            fire_adds(j, b)
            if full:
                wait_adds(bn)
            fire_gather(j + 2, bn)

        fire_gather(0, 0)
        fire_gather(1, 1)
        slot(0, 0, 2, False)
        slot(1, 1, 3, False)
        slot(2, 2, 0, True)
        slot(3, 3, 1, True)

        @pl.loop(NBUF, NST - 5, step=NBUF)
        def _(j0):
            slot(j0 + 0, 0, 2, True)
            slot(j0 + 1, 1, 3, True)
            slot(j0 + 2, 2, 0, True)
            slot(j0 + 3, 3, 1, True)

        slot(NST - 5, 0, 2, True)
        slot(NST - 4, 1, 3, True)
        slot(NST - 3, 2, 0, True)

        wait_gather()
        compute(NST - 2, 3)
        fire_adds(NST - 2, 3)
        wait_gather()
        compute(NST - 1, 0)
        fire_adds(NST - 1, 0)

        wait_adds(1)
        wait_adds(2)
        wait_adds(3)
        wait_adds(0)

        plsc.subcore_barrier()
        pltpu.sync_copy(accf.at[pl.ds(row0, ROWS_PER_TILE)],
                        pf_hbm.at[cid, pl.ds(row0, ROWS_PER_TILE)])

    return pl.kernel(
        body,
        out_type=out_type,
        mesh=_mesh,
        scratch_types=scratch,
        compiler_params=_sc_params,
    )


_aggregate = _make_aggregate_kernel(WA)


@jax.jit
def kernel(x, edge_index, W, attn_l, attn_r):
    src = edge_index[0]
    dst = edge_index[1]
    wt = W.T
    alr = jnp.stack([attn_l.reshape(OUT), attn_r.reshape(OUT)], axis=1)

    elr, feat_a, feat_b = pl.pallas_call(
        _feat_body,
        grid=(10,),
        in_specs=[
            pl.BlockSpec((1000, D), lambda i: (i, 0)),
            pl.BlockSpec((D, OUT), lambda i: (0, 0)),
            pl.BlockSpec((OUT, 2), lambda i: (0, 0)),
        ],
        out_specs=[
            pl.BlockSpec((1000, 2), lambda i: (i, 0)),
            pl.BlockSpec((1000, WA), lambda i: (i, 0)),
            pl.BlockSpec((1000, WA), lambda i: (i, 0)),
        ],
        out_shape=[
            jax.ShapeDtypeStruct((N, 2), jnp.float32),
            jax.ShapeDtypeStruct((N, WA), jnp.float32),
            jax.ShapeDtypeStruct((N, WA), jnp.float32),
        ],
    )(x, wt, alr)

    el = elr[:, 0]
    er = elr[:, 1]
    eexp = _edge_logits(el, er, src, dst)

    dst3 = dst.reshape(NW, NST, SUB)
    zf = jnp.zeros((ROWS_PER_TILE, WA), jnp.float32)
    pfa = _aggregate(feat_a, eexp, src, dst3, zf)
    pfb = _aggregate(feat_b, eexp, src, dst3, zf)

    rst = pl.pallas_call(
        _final_body,
        grid=(10,),
        in_specs=[
            pl.BlockSpec((NC, 1000, WA), lambda i: (0, i, 0)),
            pl.BlockSpec((NC, 1000, WA), lambda i: (0, i, 0)),
        ],
        out_specs=pl.BlockSpec((1000, OUT), lambda i: (i, 0)),
        out_shape=jax.ShapeDtypeStruct((N, OUT), jnp.float32),
    )(pfa, pfb)
    return rst.reshape(N, 1, OUT)
